# baseline (speedup 1.0000x reference)
import jax
import jax.numpy as jnp
from jax import lax
from jax.experimental import pallas as pl
from jax.experimental.pallas import tpu as pltpu
from jax.experimental.pallas import tpu_sc as plsc

N = 10000
E = 160000
DIN = 256
H = 8
C = 64
CW = 80
HC = H * C
EP = E + N
NC = 2
NS = 16
L = 16
K = 128
EPPAD = 172032
NCH = EPPAD // K
CH1 = EPPAD // (NS * K)
EHALF = EPPAD // 2
CHHALF = EHALF // K
CH2 = EHALF // (NS * K)
ACCROWS = 10016
ZSTRIPE = ACCROWS // NS
DUMP = 632
BN = 400

_MESH = plsc.VectorSubcoreMesh(core_axis_name="c", subcore_axis_name="s",
                               num_cores=NC, num_subcores=NS)


def _mm1_body(x_ref, wl_ref, wr_ref, xl_ref, xr_ref):
    xb = x_ref[...]
    xl_ref[...] = jnp.dot(xb, wl_ref[...], preferred_element_type=jnp.float32)
    xr_ref[...] = jnp.dot(xb, wr_ref[...], preferred_element_type=jnp.float32)


def _tc1(x, Wl1, Wr1):
    return pl.pallas_call(
        _mm1_body,
        grid=(N // BN,),
        in_specs=[
            pl.BlockSpec((BN, DIN), lambda i: (i, 0)),
            pl.BlockSpec((DIN, HC), lambda i: (0, 0)),
            pl.BlockSpec((DIN, HC), lambda i: (0, 0)),
        ],
        out_specs=[
            pl.BlockSpec((BN, HC), lambda i: (i, 0)),
            pl.BlockSpec((BN, HC), lambda i: (i, 0)),
        ],
        out_shape=[
            jax.ShapeDtypeStruct((N, HC), jnp.float32),
            jax.ShapeDtypeStruct((N, HC), jnp.float32),
        ],
    )(x, Wl1, Wr1)


def _tc2_body(acc_ref, b1_ref, wl_ref, wr_ref, xl_ref, xr_ref):
    b1 = b1_ref[...]
    xl = jnp.zeros((BN, C), jnp.float32)
    xr = jnp.zeros((BN, C), jnp.float32)
    for h in range(H):
        a = acc_ref[h][:, 0:C]
        d = acc_ref[h][:, C:C + 1]
        hp = a / (d + 1e-16) + b1[:, h * C:(h + 1) * C]
        hp = jnp.where(hp > 0, hp, jnp.exp(jnp.minimum(hp, 0.0)) - 1.0)
        xl = xl + jnp.dot(hp, wl_ref[h], preferred_element_type=jnp.float32)
        xr = xr + jnp.dot(hp, wr_ref[h], preferred_element_type=jnp.float32)
    xl_ref[...] = xl
    xr_ref[...] = xr


def _tc2(acc1, b1, Wl2, Wr2):
    return pl.pallas_call(
        _tc2_body,
        grid=(N // BN,),
        in_specs=[
            pl.BlockSpec((H, BN, CW), lambda i: (0, i, 0)),
            pl.BlockSpec((1, HC), lambda i: (0, 0)),
            pl.BlockSpec((H, C, C), lambda i: (0, 0, 0)),
            pl.BlockSpec((H, C, C), lambda i: (0, 0, 0)),
        ],
        out_specs=[
            pl.BlockSpec((BN, C), lambda i: (i, 0)),
            pl.BlockSpec((BN, C), lambda i: (i, 0)),
        ],
        out_shape=[
            jax.ShapeDtypeStruct((N, C), jnp.float32),
            jax.ShapeDtypeStruct((N, C), jnp.float32),
        ],
    )(acc1, b1.reshape(1, HC), Wl2.reshape(H, C, C), Wr2.reshape(H, C, C))


def _tc3_body(acc_ref, b2_ref, out_ref):
    acc = acc_ref[0][:, 0:C] + acc_ref[1][:, 0:C]
    den = acc_ref[0][:, C:C + 1] + acc_ref[1][:, C:C + 1]
    o = acc / (den + 1e-16) + b2_ref[...]
    m = jnp.max(o, axis=1, keepdims=True)
    out_ref[...] = o - m - jnp.log(jnp.sum(jnp.exp(o - m), axis=1,
                                           keepdims=True))


def _tc3(acc2, b2):
    return pl.pallas_call(
        _tc3_body,
        grid=(N // BN,),
        in_specs=[
            pl.BlockSpec((NC, BN, CW), lambda i: (0, i, 0)),
            pl.BlockSpec((1, C), lambda i: (0, 0)),
        ],
        out_specs=pl.BlockSpec((BN, C), lambda i: (i, 0)),
        out_shape=jax.ShapeDtypeStruct((N, C), jnp.float32),
    )(acc2, b2.reshape(1, C))


_GDN = lax.GatherDimensionNumbers(offset_dims=(), collapsed_slice_dims=(0,),
                                  start_index_map=(0,))


def _lane_sum(v, perms):
    for pm in perms:
        v = v + lax.gather(v, pm, _GDN, (1,),
                           mode=lax.GatherScatterMode.PROMISE_IN_BOUNDS)
    return v


def _zero_vmem(buf, rows, cols):
    def body(r, carry):
        for u in range(cols // L):
            buf[r, pl.ds(u * L, L)] = jnp.zeros((L,), jnp.float32)
        return carry
    lax.fori_loop(0, rows, body, 0)


def _zero_acc(s, zbuf, acc_sp):
    def body(z, carry):
        pltpu.sync_copy(zbuf, acc_sp.at[pl.ds(s * ZSTRIPE + z * K, K)])
        return carry
    nfull = ZSTRIPE // K
    lax.fori_loop(0, nfull, body, 0)
    rem = ZSTRIPE % K
    pltpu.sync_copy(zbuf.at[pl.ds(0, rem)],
                    acc_sp.at[pl.ds(s * ZSTRIPE + nfull * K, rem)])


def _dump_acc(s, first, acc_sp, acc_out):
    @pl.when(s < NS - 1)
    def _():
        r0 = s * DUMP
        pltpu.sync_copy(acc_sp.at[pl.ds(r0, DUMP)],
                        acc_out.at[first].at[pl.ds(r0, DUMP)])

    @pl.when(s == NS - 1)
    def _():
        r0 = (NS - 1) * DUMP
        pltpu.sync_copy(acc_sp.at[pl.ds(r0, N - r0)],
                        acc_out.at[first].at[pl.ds(r0, N - r0)])


def _edge_phase(ch, gbase, rowmul, h, ei3_hbm, xl_hbm, xr_hbm,
                ebuf, sidx, glidx, gridx, xlbuf, xrbuf, msgbuf,
                semi, semg, sems, acc_sp, attv, lane, perms):
    def idx_start(g, b):
        pltpu.async_copy(ei3_hbm.at[g], ebuf.at[b], semi[b])

    def idx_wait(b):
        pltpu.make_async_copy(ei3_hbm.at[0], ebuf.at[b], semi[b]).wait()

    def stage(b):
        if rowmul == 1:
            gl = ebuf.at[b].at[0]
            gr = ebuf.at[b].at[1]
        else:
            for u in range(K // L):
                glidx[b, pl.ds(u * L, L)] = (
                    ebuf[b, 0, pl.ds(u * L, L)] * rowmul + h)
                gridx[b, pl.ds(u * L, L)] = (
                    ebuf[b, 1, pl.ds(u * L, L)] * rowmul + h)
            gl = glidx.at[b]
            gr = gridx.at[b]
        pltpu.async_copy(xl_hbm.at[gl], xlbuf.at[b], semg[b])
        pltpu.async_copy(xr_hbm.at[gr], xrbuf.at[b], semg[b])

    def gather_wait(b):
        pltpu.make_async_copy(xl_hbm.at[glidx.at[b]], xlbuf.at[b],
                              semg[b]).wait()
        pltpu.make_async_copy(xr_hbm.at[gridx.at[b]], xrbuf.at[b],
                              semg[b]).wait()

    def sidx_copy(b):
        for u in range(K // L):
            sidx[b, pl.ds(u * L, L)] = ebuf[b, 2, pl.ds(u * L, L)]

    def scatter_start(b):
        pltpu.async_copy(msgbuf.at[b], acc_sp.at[sidx.at[b]], sems[b],
                         add=True)

    def scatter_wait(b):
        pltpu.make_async_copy(msgbuf.at[b], acc_sp.at[sidx.at[b]],
                              sems[b]).wait()

    def compute(b):
        def edge_body(e4, carry):
            for d in range(4):
                e = e4 * 4 + d
                av = []
                tp = []
                for u in range(4):
                    a = xlbuf[b, e, pl.ds(u * L, L)]
                    bb = xrbuf[b, e, pl.ds(u * L, L)]
                    av.append(a)
                    z = a + bb
                    lz = jnp.maximum(z, 0.2 * z)
                    tp.append(lz * attv[u])
                t = (tp[0] + tp[1]) + (tp[2] + tp[3])
                wv = jnp.exp(_lane_sum(t, perms))
                for u in range(4):
                    msgbuf[b, e, pl.ds(u * L, L)] = av[u] * wv
                msgbuf[b, e, pl.ds(C, L)] = jnp.where(lane == 0, wv, 0.0)
            return carry
        lax.fori_loop(0, K // 4, edge_body, 0)

    idx_start(gbase, 0)
    idx_wait(0)
    stage(0)
    idx_start(gbase + 1, 1)

    def body(i, carry):
        c0 = gbase + 2 * i
        more = 2 * i + 2 < ch

        idx_wait(1)
        stage(1)
        gather_wait(0)

        @pl.when(i > 0)
        def _():
            scatter_wait(0)
        sidx_copy(0)

        @pl.when(more)
        def _():
            idx_start(c0 + 2, 0)
        compute(0)
        scatter_start(0)

        @pl.when(more)
        def _():
            idx_wait(0)
            stage(0)

        gather_wait(1)

        @pl.when(i > 0)
        def _():
            scatter_wait(1)
        sidx_copy(1)

        @pl.when(2 * i + 3 < ch)
        def _():
            idx_start(c0 + 3, 1)
        compute(1)
        scatter_start(1)
        return carry

    lax.fori_loop(0, ch // 2, body, 0)
    scatter_wait(0)
    scatter_wait(1)


def _sc1_body(xl_hbm, xr_hbm, ei3_hbm, att_hbm, acc_hbm,
              attb, ebuf, sidx, glidx, gridx, xlbuf, xrbuf, msgbuf, zbuf,
              si0, si1, sg0, sg1, ss0, ss1, acc_sp):
    c = lax.axis_index("c")
    s = lax.axis_index("s")
    pltpu.sync_copy(att_hbm, attb)
    _zero_vmem(zbuf, K, CW)
    lane = lax.iota(jnp.int32, L)
    perms = [(lane ^ m).reshape(L, 1) for m in (1, 2, 4, 8)]
    semi, semg, sems = [si0, si1], [sg0, sg1], [ss0, ss1]

    def pass_body(k, carry):
        hh = c * (H // NC) + k
        _zero_acc(s, zbuf, acc_sp)
        plsc.subcore_barrier()
        attv = [attb[hh, pl.ds(u * L, L)] for u in range(4)]
        _edge_phase(CH1, s * CH1, H, hh, ei3_hbm, xl_hbm, xr_hbm,
                    ebuf, sidx, glidx, gridx, xlbuf, xrbuf, msgbuf,
                    semi, semg, sems, acc_sp, attv, lane, perms)
        plsc.subcore_barrier()
        _dump_acc(s, hh, acc_sp, acc_hbm)
        plsc.subcore_barrier()
        return carry

    lax.fori_loop(0, H // NC, pass_body, 0)


def _sc_scratch(att_rows):
    return [
        pltpu.VMEM((att_rows, C), jnp.float32),
        pltpu.VMEM((2, 3, K), jnp.int32),
        pltpu.VMEM((2, K), jnp.int32),
        pltpu.VMEM((2, K), jnp.int32),
        pltpu.VMEM((2, K), jnp.int32),
        pltpu.VMEM((2, K, C), jnp.float32),
        pltpu.VMEM((2, K, C), jnp.float32),
        pltpu.VMEM((2, K, CW), jnp.float32),
        pltpu.VMEM((K, CW), jnp.float32),
        pltpu.SemaphoreType.DMA,
        pltpu.SemaphoreType.DMA,
        pltpu.SemaphoreType.DMA,
        pltpu.SemaphoreType.DMA,
        pltpu.SemaphoreType.DMA,
        pltpu.SemaphoreType.DMA,
        pltpu.VMEM_SHARED((ACCROWS, CW), jnp.float32),
    ]


def _sc1(xl2d, xr2d, ei3, att1):
    return pl.kernel(
        _sc1_body,
        out_type=jax.ShapeDtypeStruct((H, N, CW), jnp.float32),
        mesh=_MESH,
        compiler_params=pltpu.CompilerParams(use_tc_tiling_on_sc=False),
        scratch_types=_sc_scratch(H),
    )(xl2d, xr2d, ei3, att1)


def _sc2_body(xl_hbm, xr_hbm, ei3_hbm, att_hbm, acc_hbm,
              attb, ebuf, sidx, glidx, gridx, xlbuf, xrbuf, msgbuf, zbuf,
              si0, si1, sg0, sg1, ss0, ss1, acc_sp):
    c = lax.axis_index("c")
    s = lax.axis_index("s")
    pltpu.sync_copy(att_hbm, attb)
    _zero_vmem(zbuf, K, CW)
    lane = lax.iota(jnp.int32, L)
    perms = [(lane ^ m).reshape(L, 1) for m in (1, 2, 4, 8)]
    _zero_acc(s, zbuf, acc_sp)
    plsc.subcore_barrier()
    attv = [attb[0, pl.ds(u * L, L)] for u in range(4)]
    zero = jnp.zeros((), jnp.int32)
    _edge_phase(CH2, c * CHHALF + s * CH2, 1, zero, ei3_hbm, xl_hbm, xr_hbm,
                ebuf, sidx, glidx, gridx, xlbuf, xrbuf, msgbuf,
                [si0, si1], [sg0, sg1], [ss0, ss1], acc_sp,
                attv, lane, perms)
    plsc.subcore_barrier()
    _dump_acc(s, c, acc_sp, acc_hbm)


def _sc2(xl2, xr2, ei3, att2):
    return pl.kernel(
        _sc2_body,
        out_type=jax.ShapeDtypeStruct((NC, N, CW), jnp.float32),
        mesh=_MESH,
        compiler_params=pltpu.CompilerParams(use_tc_tiling_on_sc=False),
        scratch_types=_sc_scratch(1),
    )(xl2, xr2, ei3, att2)


def kernel(x, edge_index, Wl1, Wr1, att1, b1, Wl2, Wr2, att2, b2):
    loop = jnp.arange(N, dtype=jnp.int32)
    src = jnp.concatenate([edge_index[0].astype(jnp.int32), loop])
    dst = jnp.concatenate([edge_index[1].astype(jnp.int32), loop])
    pad = EPPAD - EP
    srcg = jnp.concatenate([src, jnp.zeros((pad,), jnp.int32)])
    dstg = jnp.concatenate([dst, jnp.zeros((pad,), jnp.int32)])
    dsts = jnp.concatenate([dst, jnp.full((pad,), N, jnp.int32)])
    ei3 = jnp.stack([srcg, dstg, dsts]).reshape(3, NCH, K).transpose(1, 0, 2)

    xl1, xr1 = _tc1(x, Wl1, Wr1)
    acc1 = _sc1(xl1.reshape(N * H, C), xr1.reshape(N * H, C), ei3, att1)
    xl2, xr2 = _tc2(acc1, b1, Wl2, Wr2)
    acc2 = _sc2(xl2, xr2, ei3, att2)
    return _tc3(acc2, b2)


# speedup vs baseline: 1.9783x; 1.9783x over previous
import jax
import jax.numpy as jnp
from jax import lax
from jax.experimental import pallas as pl
from jax.experimental.pallas import tpu as pltpu
from jax.experimental.pallas import tpu_sc as plsc

N = 10000
E = 160000
DIN = 256
H = 8
C = 64
CW = 80
HC = H * C
EP = E + N
NC = 2
NS = 16
L = 16
K = 128
EPPAD = 172032
NCH = EPPAD // K
CH1 = EPPAD // (NS * K)
EHALF = EPPAD // 2
CHHALF = EHALF // K
CH2 = EHALF // (NS * K)
ACCROWS = 10016
ZSTRIPE = ACCROWS // NS
DUMP = 632
BN = 400

_MESH = plsc.VectorSubcoreMesh(core_axis_name="c", subcore_axis_name="s",
                               num_cores=NC, num_subcores=NS)


def _mm1_body(x_ref, wl_ref, wr_ref, xl_ref, xr_ref):
    xb = x_ref[...]
    xl_ref[...] = jnp.dot(xb, wl_ref[...], preferred_element_type=jnp.float32)
    xr_ref[...] = jnp.dot(xb, wr_ref[...], preferred_element_type=jnp.float32)


def _tc1(x, Wl1, Wr1):
    return pl.pallas_call(
        _mm1_body,
        grid=(N // BN,),
        in_specs=[
            pl.BlockSpec((BN, DIN), lambda i: (i, 0)),
            pl.BlockSpec((DIN, HC), lambda i: (0, 0)),
            pl.BlockSpec((DIN, HC), lambda i: (0, 0)),
        ],
        out_specs=[
            pl.BlockSpec((BN, HC), lambda i: (i, 0)),
            pl.BlockSpec((BN, HC), lambda i: (i, 0)),
        ],
        out_shape=[
            jax.ShapeDtypeStruct((N, HC), jnp.float32),
            jax.ShapeDtypeStruct((N, HC), jnp.float32),
        ],
    )(x, Wl1, Wr1)


def _tc2_body(acc_ref, b1_ref, wl_ref, wr_ref, xl_ref, xr_ref):
    b1 = b1_ref[...]
    xl = jnp.zeros((BN, C), jnp.float32)
    xr = jnp.zeros((BN, C), jnp.float32)
    for h in range(H):
        a = acc_ref[h][:, 0:C]
        d = acc_ref[h][:, C:C + 1]
        hp = a / (d + 1e-16) + b1[:, h * C:(h + 1) * C]
        hp = jnp.where(hp > 0, hp, jnp.exp(jnp.minimum(hp, 0.0)) - 1.0)
        xl = xl + jnp.dot(hp, wl_ref[h], preferred_element_type=jnp.float32)
        xr = xr + jnp.dot(hp, wr_ref[h], preferred_element_type=jnp.float32)
    xl_ref[...] = xl
    xr_ref[...] = xr


def _tc2(acc1, b1, Wl2, Wr2):
    return pl.pallas_call(
        _tc2_body,
        grid=(N // BN,),
        in_specs=[
            pl.BlockSpec((H, BN, CW), lambda i: (0, i, 0)),
            pl.BlockSpec((1, HC), lambda i: (0, 0)),
            pl.BlockSpec((H, C, C), lambda i: (0, 0, 0)),
            pl.BlockSpec((H, C, C), lambda i: (0, 0, 0)),
        ],
        out_specs=[
            pl.BlockSpec((BN, C), lambda i: (i, 0)),
            pl.BlockSpec((BN, C), lambda i: (i, 0)),
        ],
        out_shape=[
            jax.ShapeDtypeStruct((N, C), jnp.float32),
            jax.ShapeDtypeStruct((N, C), jnp.float32),
        ],
    )(acc1, b1.reshape(1, HC), Wl2.reshape(H, C, C), Wr2.reshape(H, C, C))


def _tc3_body(acc_ref, b2_ref, out_ref):
    acc = acc_ref[0][:, 0:C] + acc_ref[1][:, 0:C]
    den = acc_ref[0][:, C:C + 1] + acc_ref[1][:, C:C + 1]
    o = acc / (den + 1e-16) + b2_ref[...]
    m = jnp.max(o, axis=1, keepdims=True)
    out_ref[...] = o - m - jnp.log(jnp.sum(jnp.exp(o - m), axis=1,
                                           keepdims=True))


def _tc3(acc2, b2):
    return pl.pallas_call(
        _tc3_body,
        grid=(N // BN,),
        in_specs=[
            pl.BlockSpec((NC, BN, CW), lambda i: (0, i, 0)),
            pl.BlockSpec((1, C), lambda i: (0, 0)),
        ],
        out_specs=pl.BlockSpec((BN, C), lambda i: (i, 0)),
        out_shape=jax.ShapeDtypeStruct((N, C), jnp.float32),
    )(acc2, b2.reshape(1, C))


_GDN = lax.GatherDimensionNumbers(offset_dims=(), collapsed_slice_dims=(0,),
                                  start_index_map=(0,))


def _lane_sum(v, perms):
    for pm in perms:
        v = v + lax.gather(v, pm, _GDN, (1,),
                           mode=lax.GatherScatterMode.PROMISE_IN_BOUNDS)
    return v


def _zero_vmem(buf, rows, cols):
    def body(r, carry):
        for u in range(cols // L):
            buf[r, pl.ds(u * L, L)] = jnp.zeros((L,), jnp.float32)
        return carry
    lax.fori_loop(0, rows, body, 0)


def _zero_acc(s, zbuf, acc_sp):
    def body(z, carry):
        pltpu.sync_copy(zbuf, acc_sp.at[pl.ds(s * ZSTRIPE + z * K, K)])
        return carry
    nfull = ZSTRIPE // K
    lax.fori_loop(0, nfull, body, 0)
    rem = ZSTRIPE % K
    pltpu.sync_copy(zbuf.at[pl.ds(0, rem)],
                    acc_sp.at[pl.ds(s * ZSTRIPE + nfull * K, rem)])


def _dump_acc(s, first, acc_sp, acc_out):
    @pl.when(s < NS - 1)
    def _():
        r0 = s * DUMP
        pltpu.sync_copy(acc_sp.at[pl.ds(r0, DUMP)],
                        acc_out.at[first].at[pl.ds(r0, DUMP)])

    @pl.when(s == NS - 1)
    def _():
        r0 = (NS - 1) * DUMP
        pltpu.sync_copy(acc_sp.at[pl.ds(r0, N - r0)],
                        acc_out.at[first].at[pl.ds(r0, N - r0)])


def _edge_phase(ch, gbase, rowmul, h, ei3_hbm, xl_hbm, xr_hbm,
                ebuf, sidx, glidx, gridx, xlbuf, xrbuf, msgbuf,
                semi, semg, sems, acc_sp, attv, lane, perms):
    def idx_start(g, b):
        pltpu.async_copy(ei3_hbm.at[g], ebuf.at[b], semi[b])

    def idx_wait(b):
        pltpu.make_async_copy(ei3_hbm.at[0], ebuf.at[b], semi[b]).wait()

    def stage(b):
        if rowmul == 1:
            gl = ebuf.at[b].at[0]
            gr = ebuf.at[b].at[1]
        else:
            for u in range(K // L):
                glidx[b, pl.ds(u * L, L)] = (
                    ebuf[b, 0, pl.ds(u * L, L)] * rowmul + h)
                gridx[b, pl.ds(u * L, L)] = (
                    ebuf[b, 1, pl.ds(u * L, L)] * rowmul + h)
            gl = glidx.at[b]
            gr = gridx.at[b]
        pltpu.async_copy(xl_hbm.at[gl], xlbuf.at[b], semg[b])
        pltpu.async_copy(xr_hbm.at[gr], xrbuf.at[b], semg[b])

    def gather_wait(b):
        pltpu.make_async_copy(xl_hbm.at[glidx.at[b]], xlbuf.at[b],
                              semg[b]).wait()
        pltpu.make_async_copy(xr_hbm.at[gridx.at[b]], xrbuf.at[b],
                              semg[b]).wait()

    def sidx_copy(b):
        for u in range(K // L):
            sidx[b, pl.ds(u * L, L)] = ebuf[b, 2, pl.ds(u * L, L)]

    def scatter_start(b):
        pltpu.async_copy(msgbuf.at[b], acc_sp.at[sidx.at[b]], sems[b],
                         add=True)

    def scatter_wait(b):
        pltpu.make_async_copy(msgbuf.at[b], acc_sp.at[sidx.at[b]],
                              sems[b]).wait()

    def compute(b):
        @plsc.parallel_loop(0, K, 1, unroll=4)
        def edge_body(e):
            av = []
            t = None
            for u in range(4):
                a = xlbuf[b, e, pl.ds(u * L, L)]
                bb = xrbuf[b, e, pl.ds(u * L, L)]
                av.append(a)
                z = a + bb
                lz = jnp.maximum(z, 0.2 * z)
                t = lz * attv[u] if t is None else t + lz * attv[u]
            wv = jnp.exp(_lane_sum(t, perms))
            for u in range(4):
                msgbuf[b, e, pl.ds(u * L, L)] = av[u] * wv
            msgbuf[b, e, pl.ds(C, L)] = jnp.where(lane == 0, wv, 0.0)

    idx_start(gbase, 0)
    idx_wait(0)
    stage(0)
    idx_start(gbase + 1, 1)

    def body(i, carry):
        c0 = gbase + 2 * i
        more = 2 * i + 2 < ch

        idx_wait(1)
        stage(1)
        gather_wait(0)

        @pl.when(i > 0)
        def _():
            scatter_wait(0)
        sidx_copy(0)

        @pl.when(more)
        def _():
            idx_start(c0 + 2, 0)
        compute(0)
        scatter_start(0)

        @pl.when(more)
        def _():
            idx_wait(0)
            stage(0)

        gather_wait(1)

        @pl.when(i > 0)
        def _():
            scatter_wait(1)
        sidx_copy(1)

        @pl.when(2 * i + 3 < ch)
        def _():
            idx_start(c0 + 3, 1)
        compute(1)
        scatter_start(1)
        return carry

    lax.fori_loop(0, ch // 2, body, 0)
    scatter_wait(0)
    scatter_wait(1)


def _sc1_body(xl_hbm, xr_hbm, ei3_hbm, att_hbm, acc_hbm,
              attb, ebuf, sidx, glidx, gridx, xlbuf, xrbuf, msgbuf, zbuf,
              si0, si1, sg0, sg1, ss0, ss1, acc_sp):
    c = lax.axis_index("c")
    s = lax.axis_index("s")
    pltpu.sync_copy(att_hbm, attb)
    _zero_vmem(zbuf, K, CW)
    lane = lax.iota(jnp.int32, L)
    perms = [(lane ^ m).reshape(L, 1) for m in (1, 2, 4, 8)]
    semi, semg, sems = [si0, si1], [sg0, sg1], [ss0, ss1]

    def pass_body(k, carry):
        hh = c * (H // NC) + k
        _zero_acc(s, zbuf, acc_sp)
        plsc.subcore_barrier()
        attv = [attb[hh, pl.ds(u * L, L)] for u in range(4)]
        _edge_phase(CH1, s * CH1, H, hh, ei3_hbm, xl_hbm, xr_hbm,
                    ebuf, sidx, glidx, gridx, xlbuf, xrbuf, msgbuf,
                    semi, semg, sems, acc_sp, attv, lane, perms)
        plsc.subcore_barrier()
        _dump_acc(s, hh, acc_sp, acc_hbm)
        plsc.subcore_barrier()
        return carry

    lax.fori_loop(0, H // NC, pass_body, 0)


def _sc_scratch(att_rows):
    return [
        pltpu.VMEM((att_rows, C), jnp.float32),
        pltpu.VMEM((2, 3, K), jnp.int32),
        pltpu.VMEM((2, K), jnp.int32),
        pltpu.VMEM((2, K), jnp.int32),
        pltpu.VMEM((2, K), jnp.int32),
        pltpu.VMEM((2, K, C), jnp.float32),
        pltpu.VMEM((2, K, C), jnp.float32),
        pltpu.VMEM((2, K, CW), jnp.float32),
        pltpu.VMEM((K, CW), jnp.float32),
        pltpu.SemaphoreType.DMA,
        pltpu.SemaphoreType.DMA,
        pltpu.SemaphoreType.DMA,
        pltpu.SemaphoreType.DMA,
        pltpu.SemaphoreType.DMA,
        pltpu.SemaphoreType.DMA,
        pltpu.VMEM_SHARED((ACCROWS, CW), jnp.float32),
    ]


def _sc1(xl2d, xr2d, ei3, att1):
    return pl.kernel(
        _sc1_body,
        out_type=jax.ShapeDtypeStruct((H, N, CW), jnp.float32),
        mesh=_MESH,
        compiler_params=pltpu.CompilerParams(use_tc_tiling_on_sc=False),
        scratch_types=_sc_scratch(H),
    )(xl2d, xr2d, ei3, att1)


def _sc2_body(xl_hbm, xr_hbm, ei3_hbm, att_hbm, acc_hbm,
              attb, ebuf, sidx, glidx, gridx, xlbuf, xrbuf, msgbuf, zbuf,
              si0, si1, sg0, sg1, ss0, ss1, acc_sp):
    c = lax.axis_index("c")
    s = lax.axis_index("s")
    pltpu.sync_copy(att_hbm, attb)
    _zero_vmem(zbuf, K, CW)
    lane = lax.iota(jnp.int32, L)
    perms = [(lane ^ m).reshape(L, 1) for m in (1, 2, 4, 8)]
    _zero_acc(s, zbuf, acc_sp)
    plsc.subcore_barrier()
    attv = [attb[0, pl.ds(u * L, L)] for u in range(4)]
    zero = jnp.zeros((), jnp.int32)
    _edge_phase(CH2, c * CHHALF + s * CH2, 1, zero, ei3_hbm, xl_hbm, xr_hbm,
                ebuf, sidx, glidx, gridx, xlbuf, xrbuf, msgbuf,
                [si0, si1], [sg0, sg1], [ss0, ss1], acc_sp,
                attv, lane, perms)
    plsc.subcore_barrier()
    _dump_acc(s, c, acc_sp, acc_hbm)


def _sc2(xl2, xr2, ei3, att2):
    return pl.kernel(
        _sc2_body,
        out_type=jax.ShapeDtypeStruct((NC, N, CW), jnp.float32),
        mesh=_MESH,
        compiler_params=pltpu.CompilerParams(use_tc_tiling_on_sc=False),
        scratch_types=_sc_scratch(1),
    )(xl2, xr2, ei3, att2)


def kernel(x, edge_index, Wl1, Wr1, att1, b1, Wl2, Wr2, att2, b2):
    loop = jnp.arange(N, dtype=jnp.int32)
    src = jnp.concatenate([edge_index[0].astype(jnp.int32), loop])
    dst = jnp.concatenate([edge_index[1].astype(jnp.int32), loop])
    pad = EPPAD - EP
    srcg = jnp.concatenate([src, jnp.zeros((pad,), jnp.int32)])
    dstg = jnp.concatenate([dst, jnp.zeros((pad,), jnp.int32)])
    dsts = jnp.concatenate([dst, jnp.full((pad,), N, jnp.int32)])
    ei3 = jnp.stack([srcg, dstg, dsts]).reshape(3, NCH, K).transpose(1, 0, 2)

    xl1, xr1 = _tc1(x, Wl1, Wr1)
    acc1 = _sc1(xl1.reshape(N * H, C), xr1.reshape(N * H, C), ei3, att1)
    xl2, xr2 = _tc2(acc1, b1, Wl2, Wr2)
    acc2 = _sc2(xl2, xr2, ei3, att2)
    return _tc3(acc2, b2)


# speedup vs baseline: 1.9840x; 1.0029x over previous
import jax
import jax.numpy as jnp
from jax import lax
from jax.experimental import pallas as pl
from jax.experimental.pallas import tpu as pltpu
from jax.experimental.pallas import tpu_sc as plsc

N = 10000
E = 160000
DIN = 256
H = 8
C = 64
CW = 80
HC = H * C
EP = E + N
NC = 2
NS = 16
L = 16
K = 128
EPPAD = 172032
NCH = EPPAD // K
CH1 = EPPAD // (NS * K)
EHALF = EPPAD // 2
CHHALF = EHALF // K
CH2 = EHALF // (NS * K)
ACCROWS = 10016
ZSTRIPE = ACCROWS // NS
DUMP = 632
BN = 400

_MESH = plsc.VectorSubcoreMesh(core_axis_name="c", subcore_axis_name="s",
                               num_cores=NC, num_subcores=NS)


def _mm1_body(x_ref, wl_ref, wr_ref, xl_ref, xr_ref):
    xb = x_ref[...]
    xl_ref[...] = jnp.dot(xb, wl_ref[...], preferred_element_type=jnp.float32)
    xr_ref[...] = jnp.dot(xb, wr_ref[...], preferred_element_type=jnp.float32)


def _tc1(x, Wl1, Wr1):
    return pl.pallas_call(
        _mm1_body,
        grid=(N // BN,),
        in_specs=[
            pl.BlockSpec((BN, DIN), lambda i: (i, 0)),
            pl.BlockSpec((DIN, HC), lambda i: (0, 0)),
            pl.BlockSpec((DIN, HC), lambda i: (0, 0)),
        ],
        out_specs=[
            pl.BlockSpec((BN, HC), lambda i: (i, 0)),
            pl.BlockSpec((BN, HC), lambda i: (i, 0)),
        ],
        out_shape=[
            jax.ShapeDtypeStruct((N, HC), jnp.float32),
            jax.ShapeDtypeStruct((N, HC), jnp.float32),
        ],
    )(x, Wl1, Wr1)


def _tc2_body(acc_ref, b1_ref, wl_ref, wr_ref, xl_ref, xr_ref):
    b1 = b1_ref[...]
    xl = jnp.zeros((BN, C), jnp.float32)
    xr = jnp.zeros((BN, C), jnp.float32)
    for h in range(H):
        a = acc_ref[h][:, 0:C]
        d = acc_ref[h][:, C:C + 1]
        hp = a / (d + 1e-16) + b1[:, h * C:(h + 1) * C]
        hp = jnp.where(hp > 0, hp, jnp.exp(jnp.minimum(hp, 0.0)) - 1.0)
        xl = xl + jnp.dot(hp, wl_ref[h], preferred_element_type=jnp.float32)
        xr = xr + jnp.dot(hp, wr_ref[h], preferred_element_type=jnp.float32)
    xl_ref[...] = xl
    xr_ref[...] = xr


def _tc2(acc1, b1, Wl2, Wr2):
    return pl.pallas_call(
        _tc2_body,
        grid=(N // BN,),
        in_specs=[
            pl.BlockSpec((H, BN, CW), lambda i: (0, i, 0)),
            pl.BlockSpec((1, HC), lambda i: (0, 0)),
            pl.BlockSpec((H, C, C), lambda i: (0, 0, 0)),
            pl.BlockSpec((H, C, C), lambda i: (0, 0, 0)),
        ],
        out_specs=[
            pl.BlockSpec((BN, C), lambda i: (i, 0)),
            pl.BlockSpec((BN, C), lambda i: (i, 0)),
        ],
        out_shape=[
            jax.ShapeDtypeStruct((N, C), jnp.float32),
            jax.ShapeDtypeStruct((N, C), jnp.float32),
        ],
    )(acc1, b1.reshape(1, HC), Wl2.reshape(H, C, C), Wr2.reshape(H, C, C))


def _tc3_body(acc_ref, b2_ref, out_ref):
    acc = acc_ref[0][:, 0:C] + acc_ref[1][:, 0:C]
    den = acc_ref[0][:, C:C + 1] + acc_ref[1][:, C:C + 1]
    o = acc / (den + 1e-16) + b2_ref[...]
    m = jnp.max(o, axis=1, keepdims=True)
    out_ref[...] = o - m - jnp.log(jnp.sum(jnp.exp(o - m), axis=1,
                                           keepdims=True))


def _tc3(acc2, b2):
    return pl.pallas_call(
        _tc3_body,
        grid=(N // BN,),
        in_specs=[
            pl.BlockSpec((NC, BN, CW), lambda i: (0, i, 0)),
            pl.BlockSpec((1, C), lambda i: (0, 0)),
        ],
        out_specs=pl.BlockSpec((BN, C), lambda i: (i, 0)),
        out_shape=jax.ShapeDtypeStruct((N, C), jnp.float32),
    )(acc2, b2.reshape(1, C))


_GDN = lax.GatherDimensionNumbers(offset_dims=(), collapsed_slice_dims=(0,),
                                  start_index_map=(0,))


def _lane_sum(v, perms):
    for pm in perms:
        v = v + lax.gather(v, pm, _GDN, (1,),
                           mode=lax.GatherScatterMode.PROMISE_IN_BOUNDS)
    return v


def _zero_vmem(buf, rows, cols):
    def body(r, carry):
        for u in range(cols // L):
            buf[r, pl.ds(u * L, L)] = jnp.zeros((L,), jnp.float32)
        return carry
    lax.fori_loop(0, rows, body, 0)


def _zero_acc(s, zbuf, acc_sp):
    def body(z, carry):
        pltpu.sync_copy(zbuf, acc_sp.at[pl.ds(s * ZSTRIPE + z * K, K)])
        return carry
    nfull = ZSTRIPE // K
    lax.fori_loop(0, nfull, body, 0)
    rem = ZSTRIPE % K
    pltpu.sync_copy(zbuf.at[pl.ds(0, rem)],
                    acc_sp.at[pl.ds(s * ZSTRIPE + nfull * K, rem)])


def _dump_acc(s, first, acc_sp, acc_out):
    @pl.when(s < NS - 1)
    def _():
        r0 = s * DUMP
        pltpu.sync_copy(acc_sp.at[pl.ds(r0, DUMP)],
                        acc_out.at[first].at[pl.ds(r0, DUMP)])

    @pl.when(s == NS - 1)
    def _():
        r0 = (NS - 1) * DUMP
        pltpu.sync_copy(acc_sp.at[pl.ds(r0, N - r0)],
                        acc_out.at[first].at[pl.ds(r0, N - r0)])


def _edge_phase(ch, gbase, rowmul, h, ei3_hbm, xl_hbm, xr_hbm,
                ebuf, sidx, glidx, gridx, xlbuf, xrbuf, msgbuf,
                semi, semg, sems, acc_sp, attv, lane, perms):
    def idx_start(g, b):
        pltpu.async_copy(ei3_hbm.at[g], ebuf.at[b], semi[b])

    def idx_wait(b):
        pltpu.make_async_copy(ei3_hbm.at[0], ebuf.at[b], semi[b]).wait()

    def stage(b):
        if rowmul == 1:
            gl = ebuf.at[b].at[0]
            gr = ebuf.at[b].at[1]
        else:
            for u in range(K // L):
                glidx[b, pl.ds(u * L, L)] = (
                    ebuf[b, 0, pl.ds(u * L, L)] * rowmul + h)
                gridx[b, pl.ds(u * L, L)] = (
                    ebuf[b, 1, pl.ds(u * L, L)] * rowmul + h)
            gl = glidx.at[b]
            gr = gridx.at[b]
        pltpu.async_copy(xl_hbm.at[gl], xlbuf.at[b], semg[b])
        pltpu.async_copy(xr_hbm.at[gr], xrbuf.at[b], semg[b])

    def gather_wait(b):
        pltpu.make_async_copy(xl_hbm.at[glidx.at[b]], xlbuf.at[b],
                              semg[b]).wait()
        pltpu.make_async_copy(xr_hbm.at[gridx.at[b]], xrbuf.at[b],
                              semg[b]).wait()

    def sidx_copy(b):
        for u in range(K // L):
            sidx[b, pl.ds(u * L, L)] = ebuf[b, 2, pl.ds(u * L, L)]

    def scatter_start(b):
        pltpu.async_copy(msgbuf.at[b], acc_sp.at[sidx.at[b]], sems[b],
                         add=True)

    def scatter_wait(b):
        pltpu.make_async_copy(msgbuf.at[b], acc_sp.at[sidx.at[b]],
                              sems[b]).wait()

    def compute(b):
        @plsc.parallel_loop(0, K, 1, unroll=8)
        def edge_body(e):
            av = []
            t = None
            for u in range(4):
                a = xlbuf[b, e, pl.ds(u * L, L)]
                bb = xrbuf[b, e, pl.ds(u * L, L)]
                av.append(a)
                z = a + bb
                lz = jnp.maximum(z, 0.2 * z)
                t = lz * attv[u] if t is None else t + lz * attv[u]
            wv = jnp.exp(_lane_sum(t, perms))
            for u in range(4):
                msgbuf[b, e, pl.ds(u * L, L)] = av[u] * wv
            msgbuf[b, e, pl.ds(C, L)] = jnp.where(lane == 0, wv, 0.0)

    idx_start(gbase, 0)
    idx_wait(0)
    stage(0)
    idx_start(gbase + 1, 1)

    def body(i, carry):
        c0 = gbase + 2 * i
        more = 2 * i + 2 < ch

        idx_wait(1)
        stage(1)
        gather_wait(0)

        @pl.when(i > 0)
        def _():
            scatter_wait(0)
        sidx_copy(0)

        @pl.when(more)
        def _():
            idx_start(c0 + 2, 0)
        compute(0)
        scatter_start(0)

        @pl.when(more)
        def _():
            idx_wait(0)
            stage(0)

        gather_wait(1)

        @pl.when(i > 0)
        def _():
            scatter_wait(1)
        sidx_copy(1)

        @pl.when(2 * i + 3 < ch)
        def _():
            idx_start(c0 + 3, 1)
        compute(1)
        scatter_start(1)
        return carry

    lax.fori_loop(0, ch // 2, body, 0)
    scatter_wait(0)
    scatter_wait(1)


def _sc1_body(xl_hbm, xr_hbm, ei3_hbm, att_hbm, acc_hbm,
              attb, ebuf, sidx, glidx, gridx, xlbuf, xrbuf, msgbuf, zbuf,
              si0, si1, sg0, sg1, ss0, ss1, acc_sp):
    c = lax.axis_index("c")
    s = lax.axis_index("s")
    pltpu.sync_copy(att_hbm, attb)
    _zero_vmem(zbuf, K, CW)
    lane = lax.iota(jnp.int32, L)
    perms = [(lane ^ m).reshape(L, 1) for m in (1, 2, 4, 8)]
    semi, semg, sems = [si0, si1], [sg0, sg1], [ss0, ss1]

    def pass_body(k, carry):
        hh = c * (H // NC) + k
        _zero_acc(s, zbuf, acc_sp)
        plsc.subcore_barrier()
        attv = [attb[hh, pl.ds(u * L, L)] for u in range(4)]
        _edge_phase(CH1, s * CH1, H, hh, ei3_hbm, xl_hbm, xr_hbm,
                    ebuf, sidx, glidx, gridx, xlbuf, xrbuf, msgbuf,
                    semi, semg, sems, acc_sp, attv, lane, perms)
        plsc.subcore_barrier()
        _dump_acc(s, hh, acc_sp, acc_hbm)
        plsc.subcore_barrier()
        return carry

    lax.fori_loop(0, H // NC, pass_body, 0)


def _sc_scratch(att_rows):
    return [
        pltpu.VMEM((att_rows, C), jnp.float32),
        pltpu.VMEM((2, 3, K), jnp.int32),
        pltpu.VMEM((2, K), jnp.int32),
        pltpu.VMEM((2, K), jnp.int32),
        pltpu.VMEM((2, K), jnp.int32),
        pltpu.VMEM((2, K, C), jnp.float32),
        pltpu.VMEM((2, K, C), jnp.float32),
        pltpu.VMEM((2, K, CW), jnp.float32),
        pltpu.VMEM((K, CW), jnp.float32),
        pltpu.SemaphoreType.DMA,
        pltpu.SemaphoreType.DMA,
        pltpu.SemaphoreType.DMA,
        pltpu.SemaphoreType.DMA,
        pltpu.SemaphoreType.DMA,
        pltpu.SemaphoreType.DMA,
        pltpu.VMEM_SHARED((ACCROWS, CW), jnp.float32),
    ]


def _sc1(xl2d, xr2d, ei3, att1):
    return pl.kernel(
        _sc1_body,
        out_type=jax.ShapeDtypeStruct((H, N, CW), jnp.float32),
        mesh=_MESH,
        compiler_params=pltpu.CompilerParams(use_tc_tiling_on_sc=False),
        scratch_types=_sc_scratch(H),
    )(xl2d, xr2d, ei3, att1)


def _sc2_body(xl_hbm, xr_hbm, ei3_hbm, att_hbm, acc_hbm,
              attb, ebuf, sidx, glidx, gridx, xlbuf, xrbuf, msgbuf, zbuf,
              si0, si1, sg0, sg1, ss0, ss1, acc_sp):
    c = lax.axis_index("c")
    s = lax.axis_index("s")
    pltpu.sync_copy(att_hbm, attb)
    _zero_vmem(zbuf, K, CW)
    lane = lax.iota(jnp.int32, L)
    perms = [(lane ^ m).reshape(L, 1) for m in (1, 2, 4, 8)]
    _zero_acc(s, zbuf, acc_sp)
    plsc.subcore_barrier()
    attv = [attb[0, pl.ds(u * L, L)] for u in range(4)]
    zero = jnp.zeros((), jnp.int32)
    _edge_phase(CH2, c * CHHALF + s * CH2, 1, zero, ei3_hbm, xl_hbm, xr_hbm,
                ebuf, sidx, glidx, gridx, xlbuf, xrbuf, msgbuf,
                [si0, si1], [sg0, sg1], [ss0, ss1], acc_sp,
                attv, lane, perms)
    plsc.subcore_barrier()
    _dump_acc(s, c, acc_sp, acc_hbm)


def _sc2(xl2, xr2, ei3, att2):
    return pl.kernel(
        _sc2_body,
        out_type=jax.ShapeDtypeStruct((NC, N, CW), jnp.float32),
        mesh=_MESH,
        compiler_params=pltpu.CompilerParams(use_tc_tiling_on_sc=False),
        scratch_types=_sc_scratch(1),
    )(xl2, xr2, ei3, att2)


def kernel(x, edge_index, Wl1, Wr1, att1, b1, Wl2, Wr2, att2, b2):
    loop = jnp.arange(N, dtype=jnp.int32)
    src = jnp.concatenate([edge_index[0].astype(jnp.int32), loop])
    dst = jnp.concatenate([edge_index[1].astype(jnp.int32), loop])
    pad = EPPAD - EP
    srcg = jnp.concatenate([src, jnp.zeros((pad,), jnp.int32)])
    dstg = jnp.concatenate([dst, jnp.zeros((pad,), jnp.int32)])
    dsts = jnp.concatenate([dst, jnp.full((pad,), N, jnp.int32)])
    ei3 = jnp.stack([srcg, dstg, dsts]).reshape(3, NCH, K).transpose(1, 0, 2)

    xl1, xr1 = _tc1(x, Wl1, Wr1)
    acc1 = _sc1(xl1.reshape(N * H, C), xr1.reshape(N * H, C), ei3, att1)
    xl2, xr2 = _tc2(acc1, b1, Wl2, Wr2)
    acc2 = _sc2(xl2, xr2, ei3, att2)
    return _tc3(acc2, b2)
